# baseline (speedup 1.0000x reference)
import functools

import jax
import jax.numpy as jnp
from jax.experimental import pallas as pl
from jax.experimental.pallas import tpu as pltpu

_N_ROWS = 10000
_N_COLS = 128
_N_BLOCKS = 10
_BLOCK_ROWS = _N_ROWS // _N_BLOCKS


def _body(xb_ref, x_hbm, h_hbm, u_ref, t_ref, acc_ref, copy_sem):
    i = pl.program_id(0)

    @pl.when(i == 0)
    def _():
        pltpu.make_async_copy(x_hbm, h_hbm, copy_sem).start()

    part = jnp.sum(xb_ref[...], axis=0, keepdims=True)

    @pl.when(i == 0)
    def _():
        acc_ref[...] = part

    @pl.when(i > 0)
    def _():
        acc_ref[...] += part

    @pl.when(i == _N_BLOCKS - 1)
    def _():
        u = acc_ref[...] * (1.0 / _N_ROWS)
        u_ref[...] = u
        m = jnp.max(u, axis=1, keepdims=True)
        e = jnp.exp(u - m)
        t_ref[...] = e / jnp.sum(e, axis=1, keepdims=True)
        pltpu.make_async_copy(x_hbm, h_hbm, copy_sem).wait()


@functools.partial(jax.jit, static_argnames=())
def _fused(x):
    h, u, u_top = pl.pallas_call(
        _body,
        grid=(_N_BLOCKS,),
        in_specs=[
            pl.BlockSpec((_BLOCK_ROWS, _N_COLS), lambda i: (i, 0)),
            pl.BlockSpec(memory_space=pltpu.MemorySpace.HBM),
        ],
        out_specs=[
            pl.BlockSpec(memory_space=pltpu.MemorySpace.HBM),
            pl.BlockSpec((1, _N_COLS), lambda i: (0, 0)),
            pl.BlockSpec((1, _N_COLS), lambda i: (0, 0)),
        ],
        out_shape=[
            jax.ShapeDtypeStruct((_N_ROWS, _N_COLS), jnp.float32),
            jax.ShapeDtypeStruct((1, _N_COLS), jnp.float32),
            jax.ShapeDtypeStruct((1, _N_COLS), jnp.float32),
        ],
        scratch_shapes=[
            pltpu.VMEM((1, _N_COLS), jnp.float32),
            pltpu.SemaphoreType.DMA,
        ],
    )(x, x)
    return h, u, u_top


def kernel(x, edge_index):
    del edge_index
    return _fused(x)


# speedup vs baseline: 5.4282x; 5.4282x over previous
import functools

import jax
import jax.numpy as jnp
from jax import lax
from jax.experimental import pallas as pl
from jax.experimental.pallas import tpu as pltpu
from jax.experimental.pallas import tpu_sc as plsc

_N_ROWS = 10000
_N_COLS = 128
_LANES = 16
_N_GROUPS = _N_COLS // _LANES

_NUM_CORES = 2
_NUM_SUBCORES = 16
_N_WORKERS = _NUM_CORES * _NUM_SUBCORES

_ROWS_PER_WORKER = _N_ROWS // _N_WORKERS
_REMAINDER_BASE = _ROWS_PER_WORKER * _N_WORKERS
_N_REMAINDER = _N_ROWS - _REMAINDER_BASE

_CHUNK_ROWS = 104
_N_CHUNKS = _ROWS_PER_WORKER // _CHUNK_ROWS


def _sc_body(x_hbm, part_hbm, buf, rem_buf, out_vec, shared, sems, rem_sem):
    cid = lax.axis_index("c")
    sid = lax.axis_index("s")
    wid = sid * _NUM_CORES + cid
    base = wid * _ROWS_PER_WORKER

    def start(c, slot):
        pltpu.make_async_copy(
            x_hbm.at[pl.ds(base + c * _CHUNK_ROWS, _CHUNK_ROWS), :],
            buf.at[slot],
            sems.at[slot],
        ).start()

    def wait(c, slot):
        pltpu.make_async_copy(
            x_hbm.at[pl.ds(base + c * _CHUNK_ROWS, _CHUNK_ROWS), :],
            buf.at[slot],
            sems.at[slot],
        ).wait()

    start(0, 0)
    is_rem_worker = wid == 0

    @pl.when(is_rem_worker)
    def _():
        pltpu.make_async_copy(
            x_hbm.at[pl.ds(_REMAINDER_BASE, _N_REMAINDER), :],
            rem_buf,
            rem_sem,
        ).start()

    accs = [jnp.zeros((_LANES,), jnp.float32) for _ in range(_N_GROUPS)]
    for c in range(_N_CHUNKS):
        slot = c % 2
        if c + 1 < _N_CHUNKS:
            start(c + 1, (c + 1) % 2)
        wait(c, slot)

        def row_step(r, acc):
            return tuple(
                acc[g] + buf[slot, r, pl.ds(g * _LANES, _LANES)]
                for g in range(_N_GROUPS)
            )

        accs = list(lax.fori_loop(0, _CHUNK_ROWS, row_step, tuple(accs)))

    for g in range(_N_GROUPS):
        out_vec[pl.ds(g * _LANES, _LANES)] = accs[g]

    @pl.when(is_rem_worker)
    def _():
        pltpu.make_async_copy(
            x_hbm.at[pl.ds(_REMAINDER_BASE, _N_REMAINDER), :],
            rem_buf,
            rem_sem,
        ).wait()

        def rem_step(r, _):
            for g in range(_N_GROUPS):
                sl = pl.ds(g * _LANES, _LANES)
                out_vec[sl] = out_vec[sl] + rem_buf[r, sl]
            return 0

        lax.fori_loop(0, _N_REMAINDER, rem_step, 0)
    pltpu.sync_copy(out_vec, shared.at[sid])
    plsc.subcore_barrier()

    @pl.when(sid == 0)
    def _():
        pltpu.sync_copy(shared, buf.at[0, pl.ds(0, _NUM_SUBCORES), :])

        def comb_step(r, acc):
            return tuple(
                acc[g] + buf[0, r, pl.ds(g * _LANES, _LANES)]
                for g in range(_N_GROUPS)
            )

        tot = lax.fori_loop(
            0,
            _NUM_SUBCORES,
            comb_step,
            tuple(jnp.zeros((_LANES,), jnp.float32) for _ in range(_N_GROUPS)),
        )
        for g in range(_N_GROUPS):
            out_vec[pl.ds(g * _LANES, _LANES)] = tot[g]
        pltpu.sync_copy(out_vec, part_hbm.at[cid])


_sc_partials = pl.kernel(
    _sc_body,
    out_type=jax.ShapeDtypeStruct((_NUM_CORES, _N_COLS), jnp.float32),
    mesh=plsc.VectorSubcoreMesh(
        core_axis_name="c",
        subcore_axis_name="s",
        num_cores=_NUM_CORES,
        num_subcores=_NUM_SUBCORES,
    ),
    scratch_types=[
        pltpu.VMEM((2, _CHUNK_ROWS, _N_COLS), jnp.float32),
        pltpu.VMEM((_N_REMAINDER, _N_COLS), jnp.float32),
        pltpu.VMEM((_N_COLS,), jnp.float32),
        pltpu.VMEM_SHARED((_NUM_SUBCORES, _N_COLS), jnp.float32),
        pltpu.SemaphoreType.DMA((2,)),
        pltpu.SemaphoreType.DMA,
    ],
)


def _tc_body(p_ref, u_ref, t_ref):
    s = jnp.sum(p_ref[...], axis=0, keepdims=True)
    u = s * (1.0 / _N_ROWS)
    u_ref[...] = u
    m = jnp.max(u, axis=1, keepdims=True)
    e = jnp.exp(u - m)
    t_ref[...] = e / jnp.sum(e, axis=1, keepdims=True)


def _tc_finish(partials):
    return pl.pallas_call(
        _tc_body,
        grid=(1,),
        in_specs=[pl.BlockSpec((_NUM_CORES, _N_COLS), lambda i: (0, 0))],
        out_specs=[
            pl.BlockSpec((1, _N_COLS), lambda i: (0, 0)),
            pl.BlockSpec((1, _N_COLS), lambda i: (0, 0)),
        ],
        out_shape=[
            jax.ShapeDtypeStruct((1, _N_COLS), jnp.float32),
            jax.ShapeDtypeStruct((1, _N_COLS), jnp.float32),
        ],
    )(partials)


@jax.jit
def _run(x):
    partials = _sc_partials(x)
    u, u_top = _tc_finish(partials)
    return x, u, u_top


def kernel(x, edge_index):
    del edge_index
    return _run(x)


# speedup vs baseline: 9.5223x; 1.7542x over previous
import functools

import jax
import jax.numpy as jnp
from jax.experimental import pallas as pl
from jax.experimental.pallas import tpu as pltpu

_N_ROWS = 10000
_N_COLS = 128
_N_BLOCKS = 25
_BLOCK_ROWS = _N_ROWS // _N_BLOCKS


def _fused_body(x_ref, h_ref, u_ref, t_ref, acc_ref):
    i = pl.program_id(0)
    xb = x_ref[...]
    h_ref[...] = xb
    part = jnp.sum(xb, axis=0, keepdims=True)

    @pl.when(i == 0)
    def _():
        acc_ref[...] = part

    @pl.when(i > 0)
    def _():
        acc_ref[...] += part

    @pl.when(i == _N_BLOCKS - 1)
    def _():
        u = acc_ref[...] * (1.0 / _N_ROWS)
        u_ref[...] = u
        m = jnp.max(u, axis=1, keepdims=True)
        e = jnp.exp(u - m)
        t_ref[...] = e / jnp.sum(e, axis=1, keepdims=True)


@functools.partial(jax.jit, static_argnames=())
def _fused(x):
    h, u, u_top = pl.pallas_call(
        _fused_body,
        grid=(_N_BLOCKS,),
        in_specs=[pl.BlockSpec((_BLOCK_ROWS, _N_COLS), lambda i: (i, 0))],
        out_specs=[
            pl.BlockSpec((_BLOCK_ROWS, _N_COLS), lambda i: (i, 0)),
            pl.BlockSpec((1, _N_COLS), lambda i: (0, 0)),
            pl.BlockSpec((1, _N_COLS), lambda i: (0, 0)),
        ],
        out_shape=[
            jax.ShapeDtypeStruct((_N_ROWS, _N_COLS), jnp.float32),
            jax.ShapeDtypeStruct((1, _N_COLS), jnp.float32),
            jax.ShapeDtypeStruct((1, _N_COLS), jnp.float32),
        ],
        scratch_shapes=[pltpu.VMEM((1, _N_COLS), jnp.float32)],
    )(x)
    return h, u, u_top


def kernel(x, edge_index):
    del edge_index
    return _fused(x)


# speedup vs baseline: 20.1663x; 2.1178x over previous
import functools

import jax
import jax.numpy as jnp
from jax.experimental import pallas as pl
from jax.experimental.pallas import tpu as pltpu

_N_ROWS = 10000
_N_COLS = 128
_N_BLOCKS = 5
_BLOCK_ROWS = _N_ROWS // _N_BLOCKS


def _fused_body(x_ref, h_ref, u_ref, t_ref, acc_ref):
    i = pl.program_id(0)
    xb = x_ref[...]
    h_ref[...] = xb
    part = jnp.sum(xb, axis=0, keepdims=True)

    @pl.when(i == 0)
    def _():
        acc_ref[...] = part

    @pl.when(i > 0)
    def _():
        acc_ref[...] += part

    @pl.when(i == _N_BLOCKS - 1)
    def _():
        u = acc_ref[...] * (1.0 / _N_ROWS)
        u_ref[...] = u
        m = jnp.max(u, axis=1, keepdims=True)
        e = jnp.exp(u - m)
        t_ref[...] = e / jnp.sum(e, axis=1, keepdims=True)


@functools.partial(jax.jit, static_argnames=())
def _fused(x):
    h, u, u_top = pl.pallas_call(
        _fused_body,
        grid=(_N_BLOCKS,),
        in_specs=[pl.BlockSpec((_BLOCK_ROWS, _N_COLS), lambda i: (i, 0))],
        out_specs=[
            pl.BlockSpec((_BLOCK_ROWS, _N_COLS), lambda i: (i, 0)),
            pl.BlockSpec((1, _N_COLS), lambda i: (0, 0)),
            pl.BlockSpec((1, _N_COLS), lambda i: (0, 0)),
        ],
        out_shape=[
            jax.ShapeDtypeStruct((_N_ROWS, _N_COLS), jnp.float32),
            jax.ShapeDtypeStruct((1, _N_COLS), jnp.float32),
            jax.ShapeDtypeStruct((1, _N_COLS), jnp.float32),
        ],
        scratch_shapes=[pltpu.VMEM((1, _N_COLS), jnp.float32)],
    )(x)
    return h, u, u_top


def kernel(x, edge_index):
    del edge_index
    return _fused(x)


# speedup vs baseline: 29.0058x; 1.4383x over previous
import functools

import jax
import jax.numpy as jnp
from jax.experimental import pallas as pl
from jax.experimental.pallas import tpu as pltpu

_N_ROWS = 10000
_N_COLS = 128
_N_BLOCKS = 2
_BLOCK_ROWS = _N_ROWS // _N_BLOCKS


def _fused_body(x_ref, h_ref, u_ref, t_ref, acc_ref):
    i = pl.program_id(0)
    xb = x_ref[...]
    h_ref[...] = xb
    part = jnp.sum(xb, axis=0, keepdims=True)

    @pl.when(i == 0)
    def _():
        acc_ref[...] = part

    @pl.when(i > 0)
    def _():
        acc_ref[...] += part

    @pl.when(i == _N_BLOCKS - 1)
    def _():
        u = acc_ref[...] * (1.0 / _N_ROWS)
        u_ref[...] = u
        m = jnp.max(u, axis=1, keepdims=True)
        e = jnp.exp(u - m)
        t_ref[...] = e / jnp.sum(e, axis=1, keepdims=True)


@functools.partial(jax.jit, static_argnames=())
def _fused(x):
    h, u, u_top = pl.pallas_call(
        _fused_body,
        grid=(_N_BLOCKS,),
        in_specs=[pl.BlockSpec((_BLOCK_ROWS, _N_COLS), lambda i: (i, 0))],
        out_specs=[
            pl.BlockSpec((_BLOCK_ROWS, _N_COLS), lambda i: (i, 0)),
            pl.BlockSpec((1, _N_COLS), lambda i: (0, 0)),
            pl.BlockSpec((1, _N_COLS), lambda i: (0, 0)),
        ],
        out_shape=[
            jax.ShapeDtypeStruct((_N_ROWS, _N_COLS), jnp.float32),
            jax.ShapeDtypeStruct((1, _N_COLS), jnp.float32),
            jax.ShapeDtypeStruct((1, _N_COLS), jnp.float32),
        ],
        scratch_shapes=[pltpu.VMEM((1, _N_COLS), jnp.float32)],
    )(x)
    return h, u, u_top


def kernel(x, edge_index):
    del edge_index
    return _fused(x)
